# baseline (speedup 1.0000x reference)
import functools

import jax
import jax.numpy as jnp
from jax.experimental import pallas as pl
from jax.experimental.pallas import tpu as pltpu

D_MODEL = 768
N_HEADS = 12
D_HEAD = 64
D_LATENT = 256
D_FF = 2048
N_EXPERTS = 64
TOP_K = 2
S = 2048

T_TILE = 128
N_TILES = 96
_RBLK = 512


def _fiota(shape, dim):
    return jax.lax.broadcasted_iota(jnp.int32, shape, dim).astype(jnp.float32)


def _bdot(a, b):
    return jnp.dot(a.astype(jnp.bfloat16), b.astype(jnp.bfloat16),
                   preferred_element_type=jnp.float32)


def _ln_f32(x, scale, bias):
    m = jnp.mean(x, axis=-1, keepdims=True)
    v = jnp.mean((x - m) ** 2, axis=-1, keepdims=True)
    return (x - m) / jnp.sqrt(v + 1e-5) * scale + bias


def _k1_body(x_ref, ln1s_ref, ln1b_ref, qW_ref, kvd_ref, kvu_ref, q_ref, kv_ref):
    xh = _ln_f32(x_ref[...], ln1s_ref[...], ln1b_ref[...])
    q_ref[...] = jnp.dot(xh, qW_ref[...], preferred_element_type=jnp.float32)
    c = jnp.dot(xh, kvd_ref[...], preferred_element_type=jnp.float32)
    kv_ref[...] = jnp.dot(c, kvu_ref[...], preferred_element_type=jnp.float32)


def _k1(x, ln1s, ln1b, qW, kvd, kvu):
    return pl.pallas_call(
        _k1_body,
        out_shape=(
            jax.ShapeDtypeStruct((S, D_MODEL), jnp.float32),
            jax.ShapeDtypeStruct((S, D_MODEL), jnp.float32),
        ),
    )(x, ln1s, ln1b, qW, kvd, kvu)


def _k2_body(q_ref, kvt_ref, kv_ref, o_ref):
    q = q_ref[0]
    aw = _bdot(q, kvt_ref[0])
    aw = aw * (1.0 / (D_HEAD ** 0.5))
    m = jnp.max(aw, axis=1, keepdims=True)
    e = jnp.exp(aw - m)
    p = e / jnp.sum(e, axis=1, keepdims=True)
    o_ref[0] = _bdot(p, kv_ref[0])


def _k2(q3, kvt3, kv3):
    blk = pl.BlockSpec((1, S, D_HEAD), lambda h: (h, 0, 0))
    blkt = pl.BlockSpec((1, D_HEAD, S), lambda h: (h, 0, 0))
    return pl.pallas_call(
        _k2_body,
        grid=(N_HEADS,),
        in_specs=[blk, blkt, blk],
        out_specs=blk,
        out_shape=jax.ShapeDtypeStruct((N_HEADS, S, D_HEAD), jnp.float32),
    )(q3, kvt3, kv3)


def _k3_body(attn_ref, x_ref, oW_ref, ln2s_ref, ln2b_ref, gW_ref, gb_ref,
             h_ref, xf_ref, p0c_ref, p1c_ref, w0c_ref, w1c_ref, desc_ref):
    h = x_ref[...] + jnp.dot(attn_ref[...], oW_ref[...],
                             preferred_element_type=jnp.float32)
    h_ref[...] = h
    xf = _ln_f32(h, ln2s_ref[...], ln2b_ref[...])
    xf_ref[...] = xf
    logits = jnp.dot(xf, gW_ref[...], preferred_element_type=jnp.float32)
    logits = logits + gb_ref[...]

    lane64 = _fiota((S, N_EXPERTS), 1)
    m1 = jnp.max(logits, axis=1, keepdims=True)
    i1 = jnp.min(jnp.where(logits == m1, lane64, 1e9), axis=1, keepdims=True)
    logits2 = jnp.where(lane64 == i1, -1e30, logits)
    m2 = jnp.max(logits2, axis=1, keepdims=True)
    i2 = jnp.min(jnp.where(logits2 == m2, lane64, 1e9), axis=1, keepdims=True)
    e2 = jnp.exp(m2 - m1)
    wt1 = 1.0 / (1.0 + e2)
    wt2 = e2 / (1.0 + e2)

    tri = (_fiota((_RBLK, _RBLK), 0)
           > _fiota((_RBLK, _RBLK), 1)
           ).astype(jnp.float32)
    lane_blk = _fiota((_RBLK, N_EXPERTS), 1)
    carry = jnp.zeros((1, N_EXPERTS), jnp.float32)
    ranks = [[], []]
    for k, sel in ((0, i1), (1, i2)):
        for b in range(S // _RBLK):
            sel_b = jax.lax.slice_in_dim(sel, b * _RBLK, (b + 1) * _RBLK, axis=0)
            ob = (lane_blk == sel_b).astype(jnp.float32)
            rb = _bdot(tri, ob) + carry
            ranks[k].append(jnp.sum(rb * ob, axis=1, keepdims=True))
            carry = carry + jnp.sum(ob, axis=0, keepdims=True)
    counts = carry

    su = (_fiota((N_EXPERTS, N_EXPERTS), 0)
          < _fiota((N_EXPERTS, N_EXPERTS), 1)
          ).astype(jnp.float32)
    off = jnp.dot(counts, su, preferred_element_type=jnp.float32)
    ntiles = jnp.floor((counts + (T_TILE - 1)) * (1.0 / T_TILE))
    tilestart = jnp.dot(ntiles, su, preferred_element_type=jnp.float32)

    lane64_full = _fiota((S, N_EXPERTS), 1)
    pos = []
    for k, sel in ((0, i1), (1, i2)):
        oh = (lane64_full == sel).astype(jnp.float32)
        offsel = jnp.sum(oh * off, axis=1, keepdims=True)
        pos.append(offsel + jnp.concatenate(ranks[k], axis=0))
    p0c_ref[...] = jnp.broadcast_to(pos[0], (S, 128))
    p1c_ref[...] = jnp.broadcast_to(pos[1], (S, 128))
    w0c_ref[...] = jnp.broadcast_to(wt1, (S, 128))
    w1c_ref[...] = jnp.broadcast_to(wt2, (S, 128))

    jrow = _fiota((128, N_EXPERTS), 0)
    ts_b = jnp.broadcast_to(tilestart, (128, N_EXPERTS))
    e_j = jnp.sum((ts_b <= jrow).astype(jnp.float32), axis=1, keepdims=True) - 1.0
    lane64_128 = _fiota((128, N_EXPERTS), 1)
    oh_e = (lane64_128 == e_j).astype(jnp.float32)
    ts_e = jnp.sum(oh_e * tilestart, axis=1, keepdims=True)
    off_e = jnp.sum(oh_e * off, axis=1, keepdims=True)
    c_e = jnp.sum(oh_e * counts, axis=1, keepdims=True)
    jcol = _fiota((128, 1), 0)
    m_j = jcol - ts_e
    s_j = off_e + m_j * T_TILE
    len_j = jnp.clip(c_e - m_j * T_TILE, 0.0, float(T_TILE))
    lane = _fiota((128, 128), 1)
    descf = jnp.where(lane == 0.0, e_j,
                      jnp.where(lane == 1.0, s_j,
                                jnp.where(lane == 2.0, len_j, 0.0)))
    desc_ref[...] = descf.astype(jnp.int32)


def _k3(attn, x, oW, ln2s, ln2b, gW, gb):
    return pl.pallas_call(
        _k3_body,
        out_shape=(
            jax.ShapeDtypeStruct((S, D_MODEL), jnp.float32),
            jax.ShapeDtypeStruct((S, D_MODEL), jnp.float32),
            jax.ShapeDtypeStruct((S, 128), jnp.float32),
            jax.ShapeDtypeStruct((S, 128), jnp.float32),
            jax.ShapeDtypeStruct((S, 128), jnp.float32),
            jax.ShapeDtypeStruct((S, 128), jnp.float32),
            jax.ShapeDtypeStruct((128, 128), jnp.int32),
        ),
    )(attn, x, oW, ln2s, ln2b, gW, gb)


def _k5_body(e_ref, s_ref, l_ref, xf_ref, h_ref, p0r_ref, p1r_ref,
             w0r_ref, w1r_ref, p0c_ref, p1c_ref,
             w1_ref, b1_ref, w2_ref, b2_ref, out_ref):
    j = pl.program_id(0)

    @pl.when(j == 0)
    def _():
        out_ref[...] = h_ref[...]

    live = l_ref[j]

    @pl.when(live > 0)
    def _():
        s = s_ref[j].astype(jnp.float32)
        lf = l_ref[j].astype(jnp.float32)
        tgt_r = s + _fiota((T_TILE, 1), 0)
        valid_r = (_fiota((T_TILE, 1), 0) < lf)
        m0 = (p0r_ref[0:1, :] == tgt_r) & valid_r
        m1 = (p1r_ref[0:1, :] == tgt_r) & valid_r
        q = (m0 | m1).astype(jnp.float32)
        wt = (jnp.sum(jnp.where(m0, w0r_ref[0:1, :], 0.0), axis=1, keepdims=True)
              + jnp.sum(jnp.where(m1, w1r_ref[0:1, :], 0.0), axis=1, keepdims=True))

        tgt_c = s + _fiota((1, T_TILE), 1)
        valid_c = (_fiota((1, T_TILE), 1) < lf)
        qt0 = (p0c_ref[...] == tgt_c) & valid_c
        qt1 = (p1c_ref[...] == tgt_c) & valid_c
        qt = (qt0 | qt1).astype(jnp.float32)

        gx = _bdot(q, xf_ref[...])
        a1 = _bdot(gx, w1_ref[0])
        a1 = a1 + b1_ref[0]
        g = a1 * 0.5 * (1.0 + jax.lax.erf(a1 * (2.0 ** -0.5)))
        a2 = _bdot(g, w2_ref[0])
        a2 = (a2 + b2_ref[0]) * wt
        out_ref[...] += _bdot(qt, a2)


def _k5(desc_e, desc_s, desc_l, xf, h, p0r, p1r, w0r, w1r, p0c, p1c,
        w1, b1, w2, b2):
    full2 = lambda shape: pl.BlockSpec(shape, lambda j, *_: (0, 0))
    grid_spec = pltpu.PrefetchScalarGridSpec(
        num_scalar_prefetch=3,
        grid=(N_TILES,),
        in_specs=[
            full2((S, D_MODEL)),
            full2((S, D_MODEL)),
            full2((8, S)), full2((8, S)),
            full2((8, S)), full2((8, S)),
            full2((S, T_TILE)), full2((S, T_TILE)),
            pl.BlockSpec((1, D_MODEL, D_FF), lambda j, e, s, l: (e[j], 0, 0)),
            pl.BlockSpec((1, 1, D_FF), lambda j, e, s, l: (e[j], 0, 0)),
            pl.BlockSpec((1, D_FF, D_MODEL), lambda j, e, s, l: (e[j], 0, 0)),
            pl.BlockSpec((1, 1, D_MODEL), lambda j, e, s, l: (e[j], 0, 0)),
        ],
        out_specs=full2((S, D_MODEL)),
    )
    return pl.pallas_call(
        _k5_body,
        grid_spec=grid_spec,
        out_shape=jax.ShapeDtypeStruct((S, D_MODEL), jnp.float32),
        compiler_params=pltpu.CompilerParams(
            dimension_semantics=("arbitrary",)),
    )(desc_e, desc_s, desc_l, xf, h, p0r, p1r, w0r, w1r, p0c, p1c,
      w1, b1, w2, b2)


def kernel(x, ln1_scale, ln1_bias, q_W, kv_down_W, kv_up_W, o_W,
           ln2_scale, ln2_bias, gate_W, gate_b, w1, b1, w2, b2):
    x2 = x.reshape(S, D_MODEL)
    q, kv = _k1(x2, ln1_scale.reshape(1, -1), ln1_bias.reshape(1, -1),
                q_W, kv_down_W, kv_up_W)
    q3 = q.reshape(S, N_HEADS, D_HEAD).transpose(1, 0, 2)
    kv3 = kv.reshape(S, N_HEADS, D_HEAD).transpose(1, 0, 2)
    kvt3 = kv.reshape(S, N_HEADS, D_HEAD).transpose(1, 2, 0)
    attn3 = _k2(q3, kvt3, kv3)
    attn = attn3.transpose(1, 0, 2).reshape(S, N_HEADS * D_HEAD)

    h, xf, p0c, p1c, w0c, w1c, desc = _k3(
        attn, x2, o_W, ln2_scale.reshape(1, -1), ln2_bias.reshape(1, -1),
        gate_W, gate_b.reshape(1, -1))

    desc_e = desc[:N_TILES, 0]
    desc_s = desc[:N_TILES, 1]
    desc_l = desc[:N_TILES, 2]
    p0r = jnp.broadcast_to(p0c[:, 0][None, :], (8, S))
    p1r = jnp.broadcast_to(p1c[:, 0][None, :], (8, S))
    w0r = jnp.broadcast_to(w0c[:, 0][None, :], (8, S))
    w1r = jnp.broadcast_to(w1c[:, 0][None, :], (8, S))

    out = _k5(desc_e, desc_s, desc_l, xf, h, p0r, p1r, w0r, w1r,
              p0c[:, :T_TILE], p1c[:, :T_TILE],
              w1, b1.reshape(N_EXPERTS, 1, D_FF), w2,
              b2.reshape(N_EXPERTS, 1, D_MODEL))
    return out.reshape(x.shape)


# speedup vs baseline: 1.1892x; 1.1892x over previous
import jax
import jax.numpy as jnp
from jax.experimental import pallas as pl
from jax.experimental.pallas import tpu as pltpu

D_MODEL = 768
N_HEADS = 12
D_HEAD = 64
D_LATENT = 256
D_FF = 2048
N_EXPERTS = 64
S = 2048

T_TILE = 128
N_TILES = 96
_RBLK = 512
N_PAIR = N_HEADS // 2


def _fiota(shape, dim):
    return jax.lax.broadcasted_iota(jnp.int32, shape, dim).astype(jnp.float32)


def _bdot(a, b):
    return jnp.dot(a.astype(jnp.bfloat16), b.astype(jnp.bfloat16),
                   preferred_element_type=jnp.float32)


def _bdot_nt(a, b):
    return jax.lax.dot_general(
        a.astype(jnp.bfloat16), b.astype(jnp.bfloat16),
        (((1,), (1,)), ((), ())), preferred_element_type=jnp.float32)


def _bdot_tn(a, b):
    return jax.lax.dot_general(
        a.astype(jnp.bfloat16), b.astype(jnp.bfloat16),
        (((0,), (0,)), ((), ())), preferred_element_type=jnp.float32)


def _ln_f32(x, scale, bias):
    m = jnp.mean(x, axis=-1, keepdims=True)
    v = jnp.mean((x - m) ** 2, axis=-1, keepdims=True)
    return (x - m) / jnp.sqrt(v + 1e-5) * scale + bias


def _k123_body(x_ref, ln1s_ref, ln1b_ref, qW_ref, kvd_ref, kvu_ref, oW_ref,
               ln2s_ref, ln2b_ref, gW_ref, gb_ref,
               h_ref, xf_ref, p0c_ref, p1c_ref, w0c_ref, w1c_ref, desc_ref,
               xh_s, c_s, attn_s):
    g = pl.program_id(0)

    @pl.when(g == 0)
    def _():
        xh = _ln_f32(x_ref[...], ln1s_ref[...], ln1b_ref[...])
        xh_s[...] = xh
        c_s[...] = _bdot(xh, kvd_ref[...])

    qp = _bdot(xh_s[...], qW_ref[...])
    kvp = _bdot(c_s[...], kvu_ref[...])
    outs = []
    for half in (0, 1):
        qh = jax.lax.slice(qp, (0, half * D_HEAD), (S, (half + 1) * D_HEAD))
        kvh = jax.lax.slice(kvp, (0, half * D_HEAD), (S, (half + 1) * D_HEAD))
        ochunks = []
        for rb in range(S // _RBLK):
            qc = jax.lax.slice(qh, (rb * _RBLK, 0), ((rb + 1) * _RBLK, D_HEAD))
            aw = _bdot_nt(qc, kvh) * (1.0 / (D_HEAD ** 0.5))
            m = jnp.max(aw, axis=1, keepdims=True)
            e = jnp.exp(aw - m)
            p = e / jnp.sum(e, axis=1, keepdims=True)
            ochunks.append(_bdot(p, kvh))
        outs.append(jnp.concatenate(ochunks, axis=0))
    attn_s[g] = jnp.concatenate(outs, axis=1)

    @pl.when(g == N_PAIR - 1)
    def _():
        attn = jnp.concatenate([attn_s[i] for i in range(N_PAIR)], axis=1)
        h = x_ref[...] + jnp.dot(attn, oW_ref[...],
                                 preferred_element_type=jnp.float32)
        h_ref[...] = h
        xf = _ln_f32(h, ln2s_ref[...], ln2b_ref[...])
        xf_ref[...] = xf
        logits = jnp.dot(xf, gW_ref[...], preferred_element_type=jnp.float32)
        logits = logits + gb_ref[...]

        lane64 = _fiota((S, N_EXPERTS), 1)
        m1 = jnp.max(logits, axis=1, keepdims=True)
        i1 = jnp.min(jnp.where(logits == m1, lane64, 1e9), axis=1,
                     keepdims=True)
        logits2 = jnp.where(lane64 == i1, -1e30, logits)
        m2 = jnp.max(logits2, axis=1, keepdims=True)
        i2 = jnp.min(jnp.where(logits2 == m2, lane64, 1e9), axis=1,
                     keepdims=True)
        e2 = jnp.exp(m2 - m1)
        wt1 = 1.0 / (1.0 + e2)
        wt2 = e2 / (1.0 + e2)

        tri = (_fiota((_RBLK, _RBLK), 0) > _fiota((_RBLK, _RBLK), 1)
               ).astype(jnp.float32)
        lane_blk = _fiota((_RBLK, N_EXPERTS), 1)
        carry = jnp.zeros((1, N_EXPERTS), jnp.float32)
        ranks = [[], []]
        for k, sel in ((0, i1), (1, i2)):
            for b in range(S // _RBLK):
                sel_b = jax.lax.slice_in_dim(sel, b * _RBLK, (b + 1) * _RBLK,
                                             axis=0)
                ob = (lane_blk == sel_b).astype(jnp.float32)
                rb = _bdot(tri, ob) + carry
                ranks[k].append(jnp.sum(rb * ob, axis=1, keepdims=True))
                carry = carry + jnp.sum(ob, axis=0, keepdims=True)
        counts = carry

        su = (_fiota((N_EXPERTS, N_EXPERTS), 0)
              < _fiota((N_EXPERTS, N_EXPERTS), 1)).astype(jnp.float32)
        off = jnp.dot(counts, su, preferred_element_type=jnp.float32)
        ntiles = jnp.floor((counts + (T_TILE - 1)) * (1.0 / T_TILE))
        tilestart = jnp.dot(ntiles, su, preferred_element_type=jnp.float32)

        for k, sel, pref, wref, wt in ((0, i1, p0c_ref, w0c_ref, wt1),
                                       (1, i2, p1c_ref, w1c_ref, wt2)):
            oh = (lane64 == sel).astype(jnp.float32)
            offsel = jnp.sum(oh * off, axis=1, keepdims=True)
            posk = offsel + jnp.concatenate(ranks[k], axis=0)
            pref[...] = jnp.broadcast_to(posk, (S, 8))
            wref[...] = jnp.broadcast_to(wt, (S, 8))

        jrow = _fiota((128, N_EXPERTS), 0)
        ts_b = jnp.broadcast_to(tilestart, (128, N_EXPERTS))
        e_j = jnp.sum((ts_b <= jrow).astype(jnp.float32), axis=1,
                      keepdims=True) - 1.0
        lane64_128 = _fiota((128, N_EXPERTS), 1)
        oh_e = (lane64_128 == e_j).astype(jnp.float32)
        ts_e = jnp.sum(oh_e * tilestart, axis=1, keepdims=True)
        off_e = jnp.sum(oh_e * off, axis=1, keepdims=True)
        c_e = jnp.sum(oh_e * counts, axis=1, keepdims=True)
        jcol = _fiota((128, 1), 0)
        m_j = jcol - ts_e
        s_j = off_e + m_j * T_TILE
        len_j = jnp.clip(c_e - m_j * T_TILE, 0.0, float(T_TILE))
        lane = _fiota((128, 128), 1)
        descf = jnp.where(lane == 0.0, e_j,
                          jnp.where(lane == 1.0, s_j,
                                    jnp.where(lane == 2.0, len_j, 0.0)))
        desc_ref[...] = descf.astype(jnp.int32)


def _k123(x, ln1s, ln1b, qW, kvd, kvu, oW, ln2s, ln2b, gW, gb):
    cm = lambda shape: pl.BlockSpec(shape, lambda g: (0,) * len(shape))
    return pl.pallas_call(
        _k123_body,
        grid=(N_PAIR,),
        in_specs=[
            cm((S, D_MODEL)), cm((1, D_MODEL)), cm((1, D_MODEL)),
            pl.BlockSpec((D_MODEL, 2 * D_HEAD), lambda g: (0, g)),
            cm((D_MODEL, D_LATENT)),
            pl.BlockSpec((D_LATENT, 2 * D_HEAD), lambda g: (0, g)),
            cm((D_MODEL, D_MODEL)), cm((1, D_MODEL)), cm((1, D_MODEL)),
            cm((D_MODEL, N_EXPERTS)), cm((1, N_EXPERTS)),
        ],
        out_specs=(
            cm((S, D_MODEL)), cm((S, D_MODEL)),
            cm((S, 8)), cm((S, 8)), cm((S, 8)), cm((S, 8)),
            cm((128, 128)),
        ),
        out_shape=(
            jax.ShapeDtypeStruct((S, D_MODEL), jnp.float32),
            jax.ShapeDtypeStruct((S, D_MODEL), jnp.float32),
            jax.ShapeDtypeStruct((S, 8), jnp.float32),
            jax.ShapeDtypeStruct((S, 8), jnp.float32),
            jax.ShapeDtypeStruct((S, 8), jnp.float32),
            jax.ShapeDtypeStruct((S, 8), jnp.float32),
            jax.ShapeDtypeStruct((128, 128), jnp.int32),
        ),
        scratch_shapes=[
            pltpu.VMEM((S, D_MODEL), jnp.float32),
            pltpu.VMEM((S, D_LATENT), jnp.float32),
            pltpu.VMEM((N_PAIR, S, 2 * D_HEAD), jnp.float32),
        ],
        compiler_params=pltpu.CompilerParams(
            dimension_semantics=("arbitrary",)),
    )(x, ln1s, ln1b, qW, kvd, kvu, oW, ln2s, ln2b, gW, gb)


def _k5_body(e_ref, s_ref, l_ref, xf_ref, h_ref, p0c_ref, p1c_ref,
             w0c_ref, w1c_ref, w1_ref, b1_ref, w2_ref, b2_ref, out_ref):
    j = pl.program_id(0)

    @pl.when(j == 0)
    def _():
        out_ref[...] = h_ref[...]

    @pl.when(l_ref[j] > 0)
    def _():
        s = s_ref[j].astype(jnp.float32)
        lf = l_ref[j].astype(jnp.float32)
        tgt = s + _fiota((1, T_TILE), 1)
        valid = _fiota((1, T_TILE), 1) < lf
        p0 = jnp.broadcast_to(p0c_ref[:, 0:1], (S, T_TILE))
        p1 = jnp.broadcast_to(p1c_ref[:, 0:1], (S, T_TILE))
        qt0 = (p0 == tgt) & valid
        qt1 = (p1 == tgt) & valid
        qt = (qt0 | qt1).astype(jnp.float32)
        w0 = jnp.broadcast_to(w0c_ref[:, 0:1], (S, T_TILE))
        w1b = jnp.broadcast_to(w1c_ref[:, 0:1], (S, T_TILE))
        qtw = (jnp.where(qt0, w0, 0.0)
               + jnp.where(qt1, w1b, 0.0))

        gx = _bdot_tn(qt, xf_ref[...])
        a1 = _bdot(gx, w1_ref[0]) + b1_ref[0]
        gl = a1 * 0.5 * (1.0 + jax.lax.erf(a1 * (2.0 ** -0.5)))
        a2 = _bdot(gl, w2_ref[0]) + b2_ref[0]
        out_ref[...] += _bdot(qtw, a2)


def _k5(desc_e, desc_s, desc_l, xf, h, p0c, p1c, w0c, w1c, w1, b1, w2, b2):
    full2 = lambda shape: pl.BlockSpec(shape, lambda j, *_: (0, 0))
    grid_spec = pltpu.PrefetchScalarGridSpec(
        num_scalar_prefetch=3,
        grid=(N_TILES,),
        in_specs=[
            full2((S, D_MODEL)),
            full2((S, D_MODEL)),
            full2((S, 8)), full2((S, 8)),
            full2((S, 8)), full2((S, 8)),
            pl.BlockSpec((1, D_MODEL, D_FF), lambda j, e, s, l: (e[j], 0, 0)),
            pl.BlockSpec((1, 1, D_FF), lambda j, e, s, l: (e[j], 0, 0)),
            pl.BlockSpec((1, D_FF, D_MODEL), lambda j, e, s, l: (e[j], 0, 0)),
            pl.BlockSpec((1, 1, D_MODEL), lambda j, e, s, l: (e[j], 0, 0)),
        ],
        out_specs=full2((S, D_MODEL)),
    )
    return pl.pallas_call(
        _k5_body,
        grid_spec=grid_spec,
        out_shape=jax.ShapeDtypeStruct((S, D_MODEL), jnp.float32),
        compiler_params=pltpu.CompilerParams(
            dimension_semantics=("arbitrary",)),
    )(desc_e, desc_s, desc_l, xf, h, p0c, p1c, w0c, w1c, w1, b1, w2, b2)


def kernel(x, ln1_scale, ln1_bias, q_W, kv_down_W, kv_up_W, o_W,
           ln2_scale, ln2_bias, gate_W, gate_b, w1, b1, w2, b2):
    x2 = x.reshape(S, D_MODEL)
    h, xf, p0c, p1c, w0c, w1c, desc = _k123(
        x2, ln1_scale.reshape(1, -1), ln1_bias.reshape(1, -1),
        q_W, kv_down_W, kv_up_W, o_W,
        ln2_scale.reshape(1, -1), ln2_bias.reshape(1, -1),
        gate_W, gate_b.reshape(1, -1))

    out = _k5(desc[:N_TILES, 0], desc[:N_TILES, 1], desc[:N_TILES, 2],
              xf, h, p0c, p1c, w0c, w1c,
              w1, b1.reshape(N_EXPERTS, 1, D_FF), w2,
              b2.reshape(N_EXPERTS, 1, D_MODEL))
    return out.reshape(x.shape)


# speedup vs baseline: 1.2728x; 1.0702x over previous
import jax
import jax.numpy as jnp
from jax.experimental import pallas as pl
from jax.experimental.pallas import tpu as pltpu

D_MODEL = 768
N_HEADS = 12
D_HEAD = 64
D_LATENT = 256
D_FF = 2048
N_EXPERTS = 64
S = 2048

T_TILE = 128
N_TILES = 96
_RBLK = 512
N_PAIR = N_HEADS // 2


def _fiota(shape, dim):
    return jax.lax.broadcasted_iota(jnp.int32, shape, dim).astype(jnp.float32)


def _bdot(a, b):
    return jnp.dot(a.astype(jnp.bfloat16), b.astype(jnp.bfloat16),
                   preferred_element_type=jnp.float32)


def _bdot_nt(a, b):
    return jax.lax.dot_general(
        a.astype(jnp.bfloat16), b.astype(jnp.bfloat16),
        (((1,), (1,)), ((), ())), preferred_element_type=jnp.float32)


def _bdot_tn(a, b):
    return jax.lax.dot_general(
        a.astype(jnp.bfloat16), b.astype(jnp.bfloat16),
        (((0,), (0,)), ((), ())), preferred_element_type=jnp.float32)


def _ln_f32(x, scale, bias):
    m = jnp.mean(x, axis=-1, keepdims=True)
    v = jnp.mean((x - m) ** 2, axis=-1, keepdims=True)
    return (x - m) / jnp.sqrt(v + 1e-5) * scale + bias


def _k123_body(x_ref, ln1s_ref, ln1b_ref, qW_ref, kvd_ref, kvu_ref, oW_ref,
               ln2s_ref, ln2b_ref, gW_ref, gb_ref,
               h_ref, xf_ref, p0c_ref, p1c_ref, w0c_ref, w1c_ref, desc_ref,
               xh_s, c_s, attn_s):
    g = pl.program_id(0)

    @pl.when(g == 0)
    def _():
        xh = _ln_f32(x_ref[...], ln1s_ref[...], ln1b_ref[...])
        xh_s[...] = xh
        c_s[...] = _bdot(xh, kvd_ref[...])

    qh = _bdot(xh_s[...], qW_ref[0])
    kvh = _bdot(c_s[...], kvu_ref[0])
    ochunks = []
    for rb in range(S // _RBLK):
        qc = jax.lax.slice(qh, (rb * _RBLK, 0), ((rb + 1) * _RBLK, D_HEAD))
        aw = _bdot_nt(qc, kvh) * (1.0 / (D_HEAD ** 0.5))
        e = jnp.exp(aw)
        r = 1.0 / jnp.sum(e, axis=1, keepdims=True)
        ochunks.append(_bdot(e, kvh) * r)
    attn_s[g] = jnp.concatenate(ochunks, axis=0)

    @pl.when(g == N_HEADS - 1)
    def _():
        attn = jnp.concatenate([attn_s[i] for i in range(N_HEADS)], axis=1)
        h = x_ref[...] + jnp.dot(attn, oW_ref[...],
                                 preferred_element_type=jnp.float32)
        h_ref[...] = h
        xf = _ln_f32(h, ln2s_ref[...], ln2b_ref[...])
        xf_ref[...] = xf
        logits = jnp.dot(xf, gW_ref[...], preferred_element_type=jnp.float32)
        logits = logits + gb_ref[...]

        lane64 = _fiota((S, N_EXPERTS), 1)
        m1 = jnp.max(logits, axis=1, keepdims=True)
        i1 = jnp.min(jnp.where(logits == m1, lane64, 1e9), axis=1,
                     keepdims=True)
        logits2 = jnp.where(lane64 == i1, -1e30, logits)
        m2 = jnp.max(logits2, axis=1, keepdims=True)
        i2 = jnp.min(jnp.where(logits2 == m2, lane64, 1e9), axis=1,
                     keepdims=True)
        e2 = jnp.exp(m2 - m1)
        wt1 = 1.0 / (1.0 + e2)
        wt2 = e2 / (1.0 + e2)

        tri = (_fiota((_RBLK, _RBLK), 0) > _fiota((_RBLK, _RBLK), 1)
               ).astype(jnp.float32)
        lane_blk = _fiota((_RBLK, N_EXPERTS), 1)
        obs, carries = [], []
        carry = jnp.zeros((1, N_EXPERTS), jnp.float32)
        for sel in (i1, i2):
            for b in range(S // _RBLK):
                sel_b = jax.lax.slice_in_dim(sel, b * _RBLK, (b + 1) * _RBLK,
                                             axis=0)
                ob = (lane_blk == sel_b).astype(jnp.float32)
                obs.append(ob)
                carries.append(carry)
                carry = carry + jnp.sum(ob, axis=0, keepdims=True)
        counts = carry
        ranks = [[], []]
        nb = S // _RBLK
        for i, (ob, cb) in enumerate(zip(obs, carries)):
            rb = _bdot(tri, ob) + cb
            ranks[i // nb].append(jnp.sum(rb * ob, axis=1, keepdims=True))

        su = (_fiota((N_EXPERTS, N_EXPERTS), 0)
              < _fiota((N_EXPERTS, N_EXPERTS), 1)).astype(jnp.float32)
        off = jnp.dot(counts, su, preferred_element_type=jnp.float32)
        ntiles = jnp.floor((counts + (T_TILE - 1)) * (1.0 / T_TILE))
        tilestart = jnp.dot(ntiles, su, preferred_element_type=jnp.float32)

        for k, sel, pref, wref, wt in ((0, i1, p0c_ref, w0c_ref, wt1),
                                       (1, i2, p1c_ref, w1c_ref, wt2)):
            oh = (lane64 == sel).astype(jnp.float32)
            offsel = jnp.sum(oh * off, axis=1, keepdims=True)
            posk = offsel + jnp.concatenate(ranks[k], axis=0)
            pref[...] = jnp.broadcast_to(posk, (S, 8))
            wref[...] = jnp.broadcast_to(wt, (S, 8))

        jrow = _fiota((128, N_EXPERTS), 0)
        ts_b = jnp.broadcast_to(tilestart, (128, N_EXPERTS))
        e_j = jnp.sum((ts_b <= jrow).astype(jnp.float32), axis=1,
                      keepdims=True) - 1.0
        lane64_128 = _fiota((128, N_EXPERTS), 1)
        oh_e = (lane64_128 == e_j).astype(jnp.float32)
        ts_e = jnp.sum(oh_e * tilestart, axis=1, keepdims=True)
        off_e = jnp.sum(oh_e * off, axis=1, keepdims=True)
        c_e = jnp.sum(oh_e * counts, axis=1, keepdims=True)
        jcol = _fiota((128, 1), 0)
        m_j = jcol - ts_e
        s_j = off_e + m_j * T_TILE
        len_j = jnp.clip(c_e - m_j * T_TILE, 0.0, float(T_TILE))
        lane = _fiota((128, 128), 1)
        descf = jnp.where(lane == 0.0, e_j,
                          jnp.where(lane == 1.0, s_j,
                                    jnp.where(lane == 2.0, len_j, 0.0)))
        desc_ref[...] = descf.astype(jnp.int32)


def _k123(x, ln1s, ln1b, qW, kvd, kvu, oW, ln2s, ln2b, gW, gb):
    cm = lambda shape: pl.BlockSpec(shape, lambda g: (0,) * len(shape))
    return pl.pallas_call(
        _k123_body,
        grid=(N_HEADS,),
        in_specs=[
            cm((S, D_MODEL)), cm((1, D_MODEL)), cm((1, D_MODEL)),
            pl.BlockSpec((1, D_MODEL, D_HEAD), lambda g: (g, 0, 0)),
            cm((D_MODEL, D_LATENT)),
            pl.BlockSpec((1, D_LATENT, D_HEAD), lambda g: (g, 0, 0)),
            cm((D_MODEL, D_MODEL)), cm((1, D_MODEL)), cm((1, D_MODEL)),
            cm((D_MODEL, N_EXPERTS)), cm((1, N_EXPERTS)),
        ],
        out_specs=(
            cm((S, D_MODEL)), cm((S, D_MODEL)),
            cm((S, 8)), cm((S, 8)), cm((S, 8)), cm((S, 8)),
            cm((128, 128)),
        ),
        out_shape=(
            jax.ShapeDtypeStruct((S, D_MODEL), jnp.float32),
            jax.ShapeDtypeStruct((S, D_MODEL), jnp.float32),
            jax.ShapeDtypeStruct((S, 8), jnp.float32),
            jax.ShapeDtypeStruct((S, 8), jnp.float32),
            jax.ShapeDtypeStruct((S, 8), jnp.float32),
            jax.ShapeDtypeStruct((S, 8), jnp.float32),
            jax.ShapeDtypeStruct((128, 128), jnp.int32),
        ),
        scratch_shapes=[
            pltpu.VMEM((S, D_MODEL), jnp.float32),
            pltpu.VMEM((S, D_LATENT), jnp.float32),
            pltpu.VMEM((N_HEADS, S, D_HEAD), jnp.float32),
        ],
        compiler_params=pltpu.CompilerParams(
            dimension_semantics=("arbitrary",)),
    )(x, ln1s, ln1b, qW, kvd, kvu, oW, ln2s, ln2b, gW, gb)


def _k5_body(e_ref, s_ref, l_ref, xf_ref, h_ref, p0c_ref, p1c_ref,
             w0c_ref, w1c_ref, w1_ref, b1_ref, w2_ref, b2_ref, out_ref):
    j = pl.program_id(0)

    @pl.when(j == 0)
    def _():
        out_ref[...] = h_ref[...]

    @pl.when(l_ref[j] > 0)
    def _():
        s = s_ref[j].astype(jnp.float32)
        lf = l_ref[j].astype(jnp.float32)
        tgt = s + _fiota((1, T_TILE), 1)
        valid = _fiota((1, T_TILE), 1) < lf
        p0 = jnp.broadcast_to(p0c_ref[:, 0:1], (S, T_TILE))
        p1 = jnp.broadcast_to(p1c_ref[:, 0:1], (S, T_TILE))
        qt0 = (p0 == tgt) & valid
        qt1 = (p1 == tgt) & valid
        qt = (qt0 | qt1).astype(jnp.float32)
        w0 = jnp.broadcast_to(w0c_ref[:, 0:1], (S, T_TILE))
        w1b = jnp.broadcast_to(w1c_ref[:, 0:1], (S, T_TILE))
        qtw = (jnp.where(qt0, w0, 0.0)
               + jnp.where(qt1, w1b, 0.0))

        gx = _bdot_tn(qt, xf_ref[...])
        a1 = _bdot(gx, w1_ref[0]) + b1_ref[0]
        gl = a1 * 0.5 * (1.0 + jax.lax.erf(a1 * (2.0 ** -0.5)))
        a2 = _bdot(gl, w2_ref[0]) + b2_ref[0]
        out_ref[...] += _bdot(qtw, a2)


def _k5(desc_e, desc_s, desc_l, xf, h, p0c, p1c, w0c, w1c, w1, b1, w2, b2):
    full2 = lambda shape: pl.BlockSpec(shape, lambda j, *_: (0, 0))
    grid_spec = pltpu.PrefetchScalarGridSpec(
        num_scalar_prefetch=3,
        grid=(N_TILES,),
        in_specs=[
            full2((S, D_MODEL)),
            full2((S, D_MODEL)),
            full2((S, 8)), full2((S, 8)),
            full2((S, 8)), full2((S, 8)),
            pl.BlockSpec((1, D_MODEL, D_FF), lambda j, e, s, l: (e[j], 0, 0)),
            pl.BlockSpec((1, 1, D_FF), lambda j, e, s, l: (e[j], 0, 0)),
            pl.BlockSpec((1, D_FF, D_MODEL), lambda j, e, s, l: (e[j], 0, 0)),
            pl.BlockSpec((1, 1, D_MODEL), lambda j, e, s, l: (e[j], 0, 0)),
        ],
        out_specs=full2((S, D_MODEL)),
    )
    return pl.pallas_call(
        _k5_body,
        grid_spec=grid_spec,
        out_shape=jax.ShapeDtypeStruct((S, D_MODEL), jnp.float32),
        compiler_params=pltpu.CompilerParams(
            dimension_semantics=("arbitrary",)),
    )(desc_e, desc_s, desc_l, xf, h, p0c, p1c, w0c, w1c, w1, b1, w2, b2)


def kernel(x, ln1_scale, ln1_bias, q_W, kv_down_W, kv_up_W, o_W,
           ln2_scale, ln2_bias, gate_W, gate_b, w1, b1, w2, b2):
    x2 = x.reshape(S, D_MODEL)
    qW3 = q_W.reshape(D_MODEL, N_HEADS, D_HEAD).transpose(1, 0, 2)
    kvu3 = kv_up_W.reshape(D_LATENT, N_HEADS, D_HEAD).transpose(1, 0, 2)
    h, xf, p0c, p1c, w0c, w1c, desc = _k123(
        x2, ln1_scale.reshape(1, -1), ln1_bias.reshape(1, -1),
        qW3, kv_down_W, kvu3, o_W,
        ln2_scale.reshape(1, -1), ln2_bias.reshape(1, -1),
        gate_W, gate_b.reshape(1, -1))

    out = _k5(desc[:N_TILES, 0], desc[:N_TILES, 1], desc[:N_TILES, 2],
              xf, h, p0c, p1c, w0c, w1c,
              w1, b1.reshape(N_EXPERTS, 1, D_FF), w2,
              b2.reshape(N_EXPERTS, 1, D_MODEL))
    return out.reshape(x.shape)
